# baseline (speedup 1.0000x reference)
import functools

import jax
import jax.numpy as jnp
from jax import lax
from jax.experimental import pallas as pl
from jax.experimental.pallas import tpu as pltpu
from jax.experimental.pallas import tpu_sc as plsc

NUM_CL = 5
NUM_BINS = 256
B, C, H, W = 16, 3, 512, 512
NPIX = B * H * W
ROWS = (H * W) // 128
RCHUNK = 256
NBINS_J = NUM_CL * NUM_BINS

def _minmax_body(x_ref, mn_ref, mx_ref):
    x = x_ref[...]
    mn_ref[...] = jnp.full((1, 1, 128), jnp.min(x), jnp.float32)
    mx_ref[...] = jnp.full((1, 1, 128), jnp.max(x), jnp.float32)


def _minmax(x):
    return pl.pallas_call(
        _minmax_body,
        grid=(B,),
        in_specs=[pl.BlockSpec((1, C, ROWS, 128), lambda i: (i, 0, 0, 0))],
        out_specs=[
            pl.BlockSpec((1, 1, 128), lambda i: (i, 0, 0)),
            pl.BlockSpec((1, 1, 128), lambda i: (i, 0, 0)),
        ],
        out_shape=[
            jax.ShapeDtypeStruct((B, 1, 128), jnp.float32),
            jax.ShapeDtypeStruct((B, 1, 128), jnp.float32),
        ],
    )(x)


def _joint_body(x_ref, m_ref, mn_ref, mx_ref, o_ref):
    mn = mn_ref[0]
    mx = mx_ref[0]
    d = (mx - mn) + jnp.float32(1e-9)
    x = x_ref[0]
    nr = (x[0] - mn) / d
    ng = (x[1] - mn) / d
    nb = (x[2] - mn) / d
    gray = nr * jnp.float32(0.299) + ng * jnp.float32(0.587)
    gray = gray + nb * jnp.float32(0.114)
    q = (gray * jnp.float32(255.0)).astype(jnp.int32)
    m = m_ref[0]
    mm = jnp.maximum(jnp.maximum(jnp.maximum(jnp.maximum(m[0], m[1]), m[2]), m[3]), m[4])
    cls = jnp.full(q.shape, 4, jnp.int32)
    for c in (3, 2, 1, 0):
        cls = jnp.where(m[c] == mm, jnp.int32(c), cls)
    o_ref[0] = cls * NUM_BINS + q


def _joint(x, m, mn, mx):
    return pl.pallas_call(
        _joint_body,
        grid=(B, ROWS // RCHUNK),
        in_specs=[
            pl.BlockSpec((1, C, RCHUNK, 128), lambda i, j: (i, 0, j, 0)),
            pl.BlockSpec((1, NUM_CL, RCHUNK, 128), lambda i, j: (i, 0, j, 0)),
            pl.BlockSpec((1, 1, 128), lambda i, j: (i, 0, 0)),
            pl.BlockSpec((1, 1, 128), lambda i, j: (i, 0, 0)),
        ],
        out_specs=pl.BlockSpec((1, RCHUNK, 128), lambda i, j: (i, j, 0)),
        out_shape=jax.ShapeDtypeStruct((B, ROWS, 128), jnp.int32),
    )(x, m, mn, mx)


SC_NC = 2
SC_NS = 16
NW = SC_NC * SC_NS
NPW = NPIX // NW
CHUNK = 8192
NCHUNK = NPW // CHUNK


def _sc_hist_body(joint_hbm, out_hbm, idx0, lanehist, outbuf):
    wid = lax.axis_index("s") * SC_NC + lax.axis_index("c")
    base = wid * NPW
    laneoff = lax.iota(jnp.int32, 16) * NBINS_J
    ones = jnp.ones((16,), jnp.int32)
    zeros16 = jnp.zeros((16,), jnp.int32)

    def zero_body(i, _):
        lanehist[pl.ds(i * 16, 16)] = zeros16
        return 0

    lax.fori_loop(0, (16 * NBINS_J) // 16, zero_body, 0)

    def chunk_body(k, _):
        pltpu.sync_copy(joint_hbm.at[pl.ds(base + k * CHUNK, CHUNK)], idx0)

        def vreg_body(j, _):
            idx = idx0[pl.ds(j * 16, 16)]
            plsc.addupdate_scatter(lanehist, [idx + laneoff], ones)
            return 0

        lax.fori_loop(0, CHUNK // 16, vreg_body, 0)
        return 0

    lax.fori_loop(0, NCHUNK, chunk_body, 0)

    def red_body(i, _):
        acc = lanehist[pl.ds(i * 16, 16)]
        for l in range(1, 16):
            acc = acc + lanehist[pl.ds(l * NBINS_J + i * 16, 16)]
        outbuf[pl.ds(i * 16, 16)] = acc
        return 0

    lax.fori_loop(0, NBINS_J // 16, red_body, 0)
    pltpu.sync_copy(outbuf, out_hbm.at[wid])


def _sc_hist(joint_flat):
    mesh = plsc.VectorSubcoreMesh(
        core_axis_name="c", subcore_axis_name="s",
        num_cores=SC_NC, num_subcores=SC_NS,
    )
    f = pl.kernel(
        _sc_hist_body,
        out_type=jax.ShapeDtypeStruct((NW, NBINS_J), jnp.int32),
        mesh=mesh,
        scratch_types=[
            pltpu.VMEM((CHUNK,), jnp.int32),
            pltpu.VMEM((16 * NBINS_J,), jnp.int32),
            pltpu.VMEM((NBINS_J,), jnp.int32),
        ],
        compiler_params=pltpu.CompilerParams(needs_layout_passes=False),
    )
    return f(joint_flat)


def _final_body(p_ref, pj_ref, pi_ref, pm_ref, pim_ref, pmi_ref):
    parts = p_ref[...]
    com_i = jnp.sum(parts, axis=0)
    zeros_cnt = jnp.sum(com_i[:, 0:1])
    col = lax.broadcasted_iota(jnp.int32, (NUM_CL, NUM_BINS), 1)
    com_i = jnp.where(col == 0, zeros_cnt, com_i)
    p_joint = com_i.astype(jnp.float32) / jnp.float32(NPIX)
    p_img = jnp.sum(p_joint, axis=0, keepdims=True)
    p_mask = jnp.sum(p_joint, axis=1, keepdims=True)
    eps = jnp.float32(1e-9)
    pj_ref[...] = p_joint
    pi_ref[...] = p_img
    pm_ref[...] = p_mask
    pim_ref[...] = p_joint / (p_mask + eps)
    pmi_ref[...] = p_joint / (p_img + eps)


def _finalize(parts):
    return pl.pallas_call(
        _final_body,
        out_shape=[
            jax.ShapeDtypeStruct((NUM_CL, NUM_BINS), jnp.float32),
            jax.ShapeDtypeStruct((1, NUM_BINS), jnp.float32),
            jax.ShapeDtypeStruct((NUM_CL, 1), jnp.float32),
            jax.ShapeDtypeStruct((NUM_CL, NUM_BINS), jnp.float32),
            jax.ShapeDtypeStruct((NUM_CL, NUM_BINS), jnp.float32),
        ],
    )(parts)


def kernel(input, mask):
    x = input.reshape(B, C, ROWS, 128)
    m = mask.reshape(B, NUM_CL, ROWS, 128)
    mn, mx = _minmax(x)
    joint = _joint(x, m, mn, mx)
    parts = _sc_hist(joint.reshape(NPIX))
    p_joint, p_img, p_mask, p_img_mask, p_mask_img = _finalize(
        parts.reshape(NW, NUM_CL, NUM_BINS)
    )
    return (p_joint, p_img.reshape(NUM_BINS), p_mask.reshape(NUM_CL),
            p_img_mask, p_mask_img)


# speedup vs baseline: 1.9451x; 1.9451x over previous
import functools

import jax
import jax.numpy as jnp
from jax import lax
from jax.experimental import pallas as pl
from jax.experimental.pallas import tpu as pltpu
from jax.experimental.pallas import tpu_sc as plsc

NUM_CL = 5
NUM_BINS = 256
B, C, H, W = 16, 3, 512, 512
NPIX = B * H * W
NBINS_J = NUM_CL * NUM_BINS
BH = B // 2

SC_NC = 2
SC_NS = 16
NW = SC_NC * SC_NS
NPW = (NPIX // 2) // NW
CHUNK = NPW // 2


def _minmax_body(x_ref, mn_ref, mx_ref):
    x = x_ref[...]
    mn_ref[...] = jnp.full((1, 1, 128), jnp.min(x), jnp.float32)
    mx_ref[...] = jnp.full((1, 1, 128), jnp.max(x), jnp.float32)


def _minmax(x):
    return pl.pallas_call(
        _minmax_body,
        grid=(B,),
        in_specs=[pl.BlockSpec((1, C, H, W), lambda i: (i, 0, 0, 0))],
        out_specs=[
            pl.BlockSpec((1, 1, 128), lambda i: (i, 0, 0)),
            pl.BlockSpec((1, 1, 128), lambda i: (i, 0, 0)),
        ],
        out_shape=[
            jax.ShapeDtypeStruct((B, 1, 128), jnp.float32),
            jax.ShapeDtypeStruct((B, 1, 128), jnp.float32),
        ],
    )(x)


def _joint_body(x_ref, m_ref, mn_ref, mx_ref, o_ref):
    mn = mn_ref[0]
    mx = mx_ref[0]
    d = (mx - mn) + jnp.float32(1e-9)
    x = x_ref[0]
    nr = (x[0] - mn) / d
    ng = (x[1] - mn) / d
    nb = (x[2] - mn) / d
    gray = nr * jnp.float32(0.299) + ng * jnp.float32(0.587)
    gray = gray + nb * jnp.float32(0.114)
    q = (gray * jnp.float32(255.0)).astype(jnp.int32)
    m = m_ref[0]
    mm = jnp.maximum(jnp.maximum(jnp.maximum(jnp.maximum(m[0], m[1]), m[2]), m[3]), m[4])
    cls = jnp.full(q.shape, 4, jnp.int32)
    for c in (3, 2, 1, 0):
        cls = jnp.where(m[c] == mm, jnp.int32(c), cls)
    o_ref[0, 0] = cls * NUM_BINS + q


def _joint_half(x, m, mn, mx, img_base):
    return pl.pallas_call(
        _joint_body,
        grid=(BH, 8),
        in_specs=[
            pl.BlockSpec((1, C, 256, 128),
                         lambda i, j: (img_base + i, 0, j // 4, j % 4)),
            pl.BlockSpec((1, NUM_CL, 256, 128),
                         lambda i, j: (img_base + i, 0, j // 4, j % 4)),
            pl.BlockSpec((1, 1, 128), lambda i, j: (img_base + i, 0, 0)),
            pl.BlockSpec((1, 1, 128), lambda i, j: (img_base + i, 0, 0)),
        ],
        out_specs=pl.BlockSpec((1, 1, 256, 128), lambda i, j: (i, j, 0, 0)),
        out_shape=jax.ShapeDtypeStruct((BH, 8, 256, 128), jnp.int32),
    )(x, m, mn, mx)


def _sc_process(buf, lanehist, laneoff, ones):
    UN = 8

    def body(j, _):
        for u in range(UN):
            idx = buf[pl.ds((j * UN + u) * 16, 16)]
            plsc.addupdate_scatter(lanehist, [idx + laneoff], ones)
        return 0

    lax.fori_loop(0, CHUNK // (16 * UN), body, 0)


def _sc_hist_body(joint_hbm, out_hbm, buf0, buf1, lanehist, outbuf,
                  sem0, sem1, *, base0):
    wid = lax.axis_index("s") * SC_NC + lax.axis_index("c")
    start = base0 + wid * NPW
    laneoff = lax.iota(jnp.int32, 16) * NBINS_J
    ones = jnp.ones((16,), jnp.int32)
    zeros16 = jnp.zeros((16,), jnp.int32)

    c0 = pltpu.async_copy(joint_hbm.at[pl.ds(start, CHUNK)], buf0, sem0)
    c1 = pltpu.async_copy(joint_hbm.at[pl.ds(start + CHUNK, CHUNK)], buf1, sem1)

    def zero_body(i, _):
        for u in range(16):
            lanehist[pl.ds(i * 256 + u * 16, 16)] = zeros16
        return 0

    lax.fori_loop(0, (16 * NBINS_J) // 256, zero_body, 0)

    c0.wait()
    _sc_process(buf0, lanehist, laneoff, ones)
    c1.wait()
    _sc_process(buf1, lanehist, laneoff, ones)

    def red_body(i, _):
        acc = lanehist[pl.ds(i * 16, 16)]
        for l in range(1, 16):
            acc = acc + lanehist[pl.ds(l * NBINS_J + i * 16, 16)]
        outbuf[pl.ds(i * 16, 16)] = acc
        return 0

    lax.fori_loop(0, NBINS_J // 16, red_body, 0)
    pltpu.sync_copy(outbuf, out_hbm.at[wid])


def _sc_hist(joint_flat, base0):
    mesh = plsc.VectorSubcoreMesh(
        core_axis_name="c", subcore_axis_name="s",
        num_cores=SC_NC, num_subcores=SC_NS,
    )
    f = pl.kernel(
        functools.partial(_sc_hist_body, base0=base0),
        out_type=jax.ShapeDtypeStruct((NW, NBINS_J), jnp.int32),
        mesh=mesh,
        scratch_types=[
            pltpu.VMEM((CHUNK,), jnp.int32),
            pltpu.VMEM((CHUNK,), jnp.int32),
            pltpu.VMEM((16 * NBINS_J,), jnp.int32),
            pltpu.VMEM((NBINS_J,), jnp.int32),
            pltpu.SemaphoreType.DMA,
            pltpu.SemaphoreType.DMA,
        ],
        compiler_params=pltpu.CompilerParams(needs_layout_passes=False),
    )
    return f(joint_flat)


def _final_body(pa_ref, pb_ref, pj_ref, pi_ref, pm_ref, pim_ref, pmi_ref):
    com_i = jnp.sum(pa_ref[...], axis=0) + jnp.sum(pb_ref[...], axis=0)
    zeros_cnt = jnp.sum(com_i[:, 0:1])
    col = lax.broadcasted_iota(jnp.int32, (NUM_CL, NUM_BINS), 1)
    com_i = jnp.where(col == 0, zeros_cnt, com_i)
    p_joint = com_i.astype(jnp.float32) / jnp.float32(NPIX)
    p_img = jnp.sum(p_joint, axis=0, keepdims=True)
    p_mask = jnp.sum(p_joint, axis=1, keepdims=True)
    eps = jnp.float32(1e-9)
    pj_ref[...] = p_joint
    pi_ref[...] = p_img
    pm_ref[...] = p_mask
    pim_ref[...] = p_joint / (p_mask + eps)
    pmi_ref[...] = p_joint / (p_img + eps)


def _finalize(parts_a, parts_b):
    return pl.pallas_call(
        _final_body,
        out_shape=[
            jax.ShapeDtypeStruct((NUM_CL, NUM_BINS), jnp.float32),
            jax.ShapeDtypeStruct((1, NUM_BINS), jnp.float32),
            jax.ShapeDtypeStruct((NUM_CL, 1), jnp.float32),
            jax.ShapeDtypeStruct((NUM_CL, NUM_BINS), jnp.float32),
            jax.ShapeDtypeStruct((NUM_CL, NUM_BINS), jnp.float32),
        ],
    )(parts_a, parts_b)


def kernel(input, mask):
    mn, mx = _minmax(input)
    joint_a = _joint_half(input, mask, mn, mx, 0)
    parts_a = _sc_hist(joint_a.reshape(NPIX // 2), 0)
    joint_b = _joint_half(input, mask, mn, mx, BH)
    parts_b = _sc_hist(joint_b.reshape(NPIX // 2), 0)
    p_joint, p_img, p_mask, p_img_mask, p_mask_img = _finalize(
        parts_a.reshape(NW, NUM_CL, NUM_BINS),
        parts_b.reshape(NW, NUM_CL, NUM_BINS),
    )
    return (p_joint, p_img.reshape(NUM_BINS), p_mask.reshape(NUM_CL),
            p_img_mask, p_mask_img)


# speedup vs baseline: 2.2340x; 1.1485x over previous
import functools

import jax
import jax.numpy as jnp
from jax import lax
from jax.experimental import pallas as pl
from jax.experimental.pallas import tpu as pltpu
from jax.experimental.pallas import tpu_sc as plsc

NUM_CL = 5
NUM_BINS = 256
B, C, H, W = 16, 3, 512, 512
NPIX = B * H * W
NBINS_J = NUM_CL * NUM_BINS
BH = B // 2

SC_NC = 2
SC_NS = 16
NW = SC_NC * SC_NS
NPW = (NPIX // 2) // NW
CHUNK = NPW // 2


def _minmax_body(x_ref, mn_ref, mx_ref):
    x = x_ref[...]
    mn_ref[...] = jnp.full((1, 1, 128), jnp.min(x), jnp.float32)
    mx_ref[...] = jnp.full((1, 1, 128), jnp.max(x), jnp.float32)


def _minmax(x):
    return pl.pallas_call(
        _minmax_body,
        grid=(B,),
        in_specs=[pl.BlockSpec((1, C, H, W), lambda i: (i, 0, 0, 0))],
        out_specs=[
            pl.BlockSpec((1, 1, 128), lambda i: (i, 0, 0)),
            pl.BlockSpec((1, 1, 128), lambda i: (i, 0, 0)),
        ],
        out_shape=[
            jax.ShapeDtypeStruct((B, 1, 128), jnp.float32),
            jax.ShapeDtypeStruct((B, 1, 128), jnp.float32),
        ],
    )(x)


def _joint_body(x_ref, m_ref, mn_ref, mx_ref, o_ref):
    mn = mn_ref[0]
    mx = mx_ref[0]
    d = (mx - mn) + jnp.float32(1e-9)
    x = x_ref[0]
    nr = (x[0] - mn) / d
    ng = (x[1] - mn) / d
    nb = (x[2] - mn) / d
    gray = nr * jnp.float32(0.299) + ng * jnp.float32(0.587)
    gray = gray + nb * jnp.float32(0.114)
    q = (gray * jnp.float32(255.0)).astype(jnp.int32)
    m = m_ref[0]
    mm = jnp.maximum(jnp.maximum(jnp.maximum(jnp.maximum(m[0], m[1]), m[2]), m[3]), m[4])
    cls = jnp.full(q.shape, 4, jnp.int32)
    for c in (3, 2, 1, 0):
        cls = jnp.where(m[c] == mm, jnp.int32(c), cls)
    o_ref[0, 0] = cls * NUM_BINS + q


def _joint_half(x, m, mn, mx, img_base):
    return pl.pallas_call(
        _joint_body,
        grid=(BH, 8),
        in_specs=[
            pl.BlockSpec((1, C, 256, 128),
                         lambda i, j: (img_base + i, 0, j // 4, j % 4)),
            pl.BlockSpec((1, NUM_CL, 256, 128),
                         lambda i, j: (img_base + i, 0, j // 4, j % 4)),
            pl.BlockSpec((1, 1, 128), lambda i, j: (img_base + i, 0, 0)),
            pl.BlockSpec((1, 1, 128), lambda i, j: (img_base + i, 0, 0)),
        ],
        out_specs=pl.BlockSpec((1, 1, 256, 128), lambda i, j: (i, j, 0, 0)),
        out_shape=jax.ShapeDtypeStruct((BH, 8, 256, 128), jnp.int32),
    )(x, m, mn, mx)


def _sc_process(buf, lanehist, laneoff, ones):
    @plsc.parallel_loop(0, CHUNK // 16, 1, unroll=8)
    def _(j):
        idx = buf[pl.ds(j * 16, 16)]
        plsc.addupdate_scatter(lanehist, [idx + laneoff], ones)


def _sc_hist_body(joint_hbm, out_hbm, buf0, buf1, lanehist, outbuf,
                  sem0, sem1, *, base0):
    wid = lax.axis_index("s") * SC_NC + lax.axis_index("c")
    start = base0 + wid * NPW
    laneoff = lax.iota(jnp.int32, 16) * NBINS_J
    ones = jnp.ones((16,), jnp.int32)
    zeros16 = jnp.zeros((16,), jnp.int32)

    c0 = pltpu.async_copy(joint_hbm.at[pl.ds(start, CHUNK)], buf0, sem0)
    c1 = pltpu.async_copy(joint_hbm.at[pl.ds(start + CHUNK, CHUNK)], buf1, sem1)

    @plsc.parallel_loop(0, (16 * NBINS_J) // 16, 1, unroll=8)
    def _(i):
        lanehist[pl.ds(i * 16, 16)] = zeros16

    c0.wait()
    _sc_process(buf0, lanehist, laneoff, ones)
    c1.wait()
    _sc_process(buf1, lanehist, laneoff, ones)

    @plsc.parallel_loop(0, NBINS_J // 16, 1, unroll=2)
    def _(i):
        acc = lanehist[pl.ds(i * 16, 16)]
        for l in range(1, 16):
            acc = acc + lanehist[pl.ds(l * NBINS_J + i * 16, 16)]
        outbuf[pl.ds(i * 16, 16)] = acc
    pltpu.sync_copy(outbuf, out_hbm.at[wid])


def _sc_hist(joint_flat, base0):
    mesh = plsc.VectorSubcoreMesh(
        core_axis_name="c", subcore_axis_name="s",
        num_cores=SC_NC, num_subcores=SC_NS,
    )
    f = pl.kernel(
        functools.partial(_sc_hist_body, base0=base0),
        out_type=jax.ShapeDtypeStruct((NW, NBINS_J), jnp.int32),
        mesh=mesh,
        scratch_types=[
            pltpu.VMEM((CHUNK,), jnp.int32),
            pltpu.VMEM((CHUNK,), jnp.int32),
            pltpu.VMEM((16 * NBINS_J,), jnp.int32),
            pltpu.VMEM((NBINS_J,), jnp.int32),
            pltpu.SemaphoreType.DMA,
            pltpu.SemaphoreType.DMA,
        ],
        compiler_params=pltpu.CompilerParams(needs_layout_passes=False),
    )
    return f(joint_flat)


def _final_body(pa_ref, pb_ref, pj_ref, pi_ref, pm_ref, pim_ref, pmi_ref):
    com_i = jnp.sum(pa_ref[...], axis=0) + jnp.sum(pb_ref[...], axis=0)
    zeros_cnt = jnp.sum(com_i[:, 0:1])
    col = lax.broadcasted_iota(jnp.int32, (NUM_CL, NUM_BINS), 1)
    com_i = jnp.where(col == 0, zeros_cnt, com_i)
    p_joint = com_i.astype(jnp.float32) / jnp.float32(NPIX)
    p_img = jnp.sum(p_joint, axis=0, keepdims=True)
    p_mask = jnp.sum(p_joint, axis=1, keepdims=True)
    eps = jnp.float32(1e-9)
    pj_ref[...] = p_joint
    pi_ref[...] = p_img
    pm_ref[...] = p_mask
    pim_ref[...] = p_joint / (p_mask + eps)
    pmi_ref[...] = p_joint / (p_img + eps)


def _finalize(parts_a, parts_b):
    return pl.pallas_call(
        _final_body,
        out_shape=[
            jax.ShapeDtypeStruct((NUM_CL, NUM_BINS), jnp.float32),
            jax.ShapeDtypeStruct((1, NUM_BINS), jnp.float32),
            jax.ShapeDtypeStruct((NUM_CL, 1), jnp.float32),
            jax.ShapeDtypeStruct((NUM_CL, NUM_BINS), jnp.float32),
            jax.ShapeDtypeStruct((NUM_CL, NUM_BINS), jnp.float32),
        ],
    )(parts_a, parts_b)


def kernel(input, mask):
    mn, mx = _minmax(input)
    joint_a = _joint_half(input, mask, mn, mx, 0)
    parts_a = _sc_hist(joint_a.reshape(NPIX // 2), 0)
    joint_b = _joint_half(input, mask, mn, mx, BH)
    parts_b = _sc_hist(joint_b.reshape(NPIX // 2), 0)
    p_joint, p_img, p_mask, p_img_mask, p_mask_img = _finalize(
        parts_a.reshape(NW, NUM_CL, NUM_BINS),
        parts_b.reshape(NW, NUM_CL, NUM_BINS),
    )
    return (p_joint, p_img.reshape(NUM_BINS), p_mask.reshape(NUM_CL),
            p_img_mask, p_mask_img)


# speedup vs baseline: 2.3004x; 1.0297x over previous
import functools

import jax
import jax.numpy as jnp
from jax import lax
from jax.experimental import pallas as pl
from jax.experimental.pallas import tpu as pltpu
from jax.experimental.pallas import tpu_sc as plsc

NUM_CL = 5
NUM_BINS = 256
B, C, H, W = 16, 3, 512, 512
NPIX = B * H * W
NBINS_J = NUM_CL * NUM_BINS
BH = B // 2

SC_NC = 2
SC_NS = 16
NW = SC_NC * SC_NS
NPW = (NPIX // 4) // NW
CHUNK = NPW // 2


def _minmax_body(x_ref, mn_ref, mx_ref):
    x = x_ref[...]
    mn_ref[...] = jnp.full((1, 1, 128), jnp.min(x), jnp.float32)
    mx_ref[...] = jnp.full((1, 1, 128), jnp.max(x), jnp.float32)


def _minmax(x):
    return pl.pallas_call(
        _minmax_body,
        grid=(B,),
        in_specs=[pl.BlockSpec((1, C, H, W), lambda i: (i, 0, 0, 0))],
        out_specs=[
            pl.BlockSpec((1, 1, 128), lambda i: (i, 0, 0)),
            pl.BlockSpec((1, 1, 128), lambda i: (i, 0, 0)),
        ],
        out_shape=[
            jax.ShapeDtypeStruct((B, 1, 128), jnp.float32),
            jax.ShapeDtypeStruct((B, 1, 128), jnp.float32),
        ],
    )(x)


def _joint_body(x_ref, m_ref, mn_ref, mx_ref, o_ref):
    mn = mn_ref[0]
    mx = mx_ref[0]
    d = (mx - mn) + jnp.float32(1e-9)
    x = x_ref[0]
    nr = (x[0] - mn) / d
    ng = (x[1] - mn) / d
    nb = (x[2] - mn) / d
    gray = nr * jnp.float32(0.299) + ng * jnp.float32(0.587)
    gray = gray + nb * jnp.float32(0.114)
    q = (gray * jnp.float32(255.0)).astype(jnp.int32)
    m = m_ref[0]
    mm = jnp.maximum(jnp.maximum(jnp.maximum(jnp.maximum(m[0], m[1]), m[2]), m[3]), m[4])
    cls = jnp.full(q.shape, 4, jnp.int32)
    for c in (3, 2, 1, 0):
        cls = jnp.where(m[c] == mm, jnp.int32(c), cls)
    joint = cls * NUM_BINS + q
    o_ref[0, 0] = jnp.bitwise_or(joint[0:128], lax.shift_left(joint[128:256], 16))


def _joint_half(x, m, mn, mx, img_base):
    return pl.pallas_call(
        _joint_body,
        grid=(BH, 8),
        in_specs=[
            pl.BlockSpec((1, C, 256, 128),
                         lambda i, j: (img_base + i, 0, j // 4, j % 4)),
            pl.BlockSpec((1, NUM_CL, 256, 128),
                         lambda i, j: (img_base + i, 0, j // 4, j % 4)),
            pl.BlockSpec((1, 1, 128), lambda i, j: (img_base + i, 0, 0)),
            pl.BlockSpec((1, 1, 128), lambda i, j: (img_base + i, 0, 0)),
        ],
        out_specs=pl.BlockSpec((1, 1, 128, 128), lambda i, j: (i, j, 0, 0)),
        out_shape=jax.ShapeDtypeStruct((BH, 8, 128, 128), jnp.int32),
    )(x, m, mn, mx)


def _sc_process(buf, lanehist, laneoff, ones):
    @plsc.parallel_loop(0, CHUNK // 16, 1, unroll=8)
    def _(j):
        v = buf[pl.ds(j * 16, 16)]
        lo = jnp.bitwise_and(v, jnp.int32(0xFFFF))
        hi = lax.shift_right_logical(v, 16)
        plsc.addupdate_scatter(lanehist, [lo + laneoff], ones)
        plsc.addupdate_scatter(lanehist, [hi + laneoff], ones)


def _sc_hist_body(joint_hbm, out_hbm, buf0, buf1, lanehist, outbuf,
                  sem0, sem1, *, base0):
    wid = lax.axis_index("s") * SC_NC + lax.axis_index("c")
    start = base0 + wid * NPW
    laneoff = lax.iota(jnp.int32, 16) * NBINS_J
    ones = jnp.ones((16,), jnp.int32)
    zeros16 = jnp.zeros((16,), jnp.int32)

    c0 = pltpu.async_copy(joint_hbm.at[pl.ds(start, CHUNK)], buf0, sem0)
    c1 = pltpu.async_copy(joint_hbm.at[pl.ds(start + CHUNK, CHUNK)], buf1, sem1)

    @plsc.parallel_loop(0, (16 * NBINS_J) // 16, 1, unroll=8)
    def _(i):
        lanehist[pl.ds(i * 16, 16)] = zeros16

    c0.wait()
    _sc_process(buf0, lanehist, laneoff, ones)
    c1.wait()
    _sc_process(buf1, lanehist, laneoff, ones)

    @plsc.parallel_loop(0, NBINS_J // 16, 1, unroll=2)
    def _(i):
        acc = lanehist[pl.ds(i * 16, 16)]
        for l in range(1, 16):
            acc = acc + lanehist[pl.ds(l * NBINS_J + i * 16, 16)]
        outbuf[pl.ds(i * 16, 16)] = acc
    pltpu.sync_copy(outbuf, out_hbm.at[wid])


def _sc_hist(joint_flat, base0):
    mesh = plsc.VectorSubcoreMesh(
        core_axis_name="c", subcore_axis_name="s",
        num_cores=SC_NC, num_subcores=SC_NS,
    )
    f = pl.kernel(
        functools.partial(_sc_hist_body, base0=base0),
        out_type=jax.ShapeDtypeStruct((NW, NBINS_J), jnp.int32),
        mesh=mesh,
        scratch_types=[
            pltpu.VMEM((CHUNK,), jnp.int32),
            pltpu.VMEM((CHUNK,), jnp.int32),
            pltpu.VMEM((16 * NBINS_J,), jnp.int32),
            pltpu.VMEM((NBINS_J,), jnp.int32),
            pltpu.SemaphoreType.DMA,
            pltpu.SemaphoreType.DMA,
        ],
        compiler_params=pltpu.CompilerParams(needs_layout_passes=False),
    )
    return f(joint_flat)


def _final_body(pa_ref, pb_ref, pj_ref, pi_ref, pm_ref, pim_ref, pmi_ref):
    com_i = jnp.sum(pa_ref[...], axis=0) + jnp.sum(pb_ref[...], axis=0)
    zeros_cnt = jnp.sum(com_i[:, 0:1])
    col = lax.broadcasted_iota(jnp.int32, (NUM_CL, NUM_BINS), 1)
    com_i = jnp.where(col == 0, zeros_cnt, com_i)
    p_joint = com_i.astype(jnp.float32) / jnp.float32(NPIX)
    p_img = jnp.sum(p_joint, axis=0, keepdims=True)
    p_mask = jnp.sum(p_joint, axis=1, keepdims=True)
    eps = jnp.float32(1e-9)
    pj_ref[...] = p_joint
    pi_ref[...] = p_img
    pm_ref[...] = p_mask
    pim_ref[...] = p_joint / (p_mask + eps)
    pmi_ref[...] = p_joint / (p_img + eps)


def _finalize(parts_a, parts_b):
    return pl.pallas_call(
        _final_body,
        out_shape=[
            jax.ShapeDtypeStruct((NUM_CL, NUM_BINS), jnp.float32),
            jax.ShapeDtypeStruct((1, NUM_BINS), jnp.float32),
            jax.ShapeDtypeStruct((NUM_CL, 1), jnp.float32),
            jax.ShapeDtypeStruct((NUM_CL, NUM_BINS), jnp.float32),
            jax.ShapeDtypeStruct((NUM_CL, NUM_BINS), jnp.float32),
        ],
    )(parts_a, parts_b)


def kernel(input, mask):
    mn, mx = _minmax(input)
    joint_a = _joint_half(input, mask, mn, mx, 0)
    parts_a = _sc_hist(joint_a.reshape(NPIX // 4), 0)
    joint_b = _joint_half(input, mask, mn, mx, BH)
    parts_b = _sc_hist(joint_b.reshape(NPIX // 4), 0)
    p_joint, p_img, p_mask, p_img_mask, p_mask_img = _finalize(
        parts_a.reshape(NW, NUM_CL, NUM_BINS),
        parts_b.reshape(NW, NUM_CL, NUM_BINS),
    )
    return (p_joint, p_img.reshape(NUM_BINS), p_mask.reshape(NUM_CL),
            p_img_mask, p_mask_img)


# speedup vs baseline: 3.2577x; 1.4161x over previous
import functools

import jax
import jax.numpy as jnp
from jax import lax
from jax.experimental import pallas as pl
from jax.experimental.pallas import tpu as pltpu
from jax.experimental.pallas import tpu_sc as plsc

NUM_CL = 5
NUM_BINS = 256
B, C, H, W = 16, 3, 512, 512
NPIX = B * H * W
NBINS_J = NUM_CL * NUM_BINS
BH = B // 2

SC_NC = 2
SC_NS = 16
NW = SC_NC * SC_NS
NPW = (NPIX // 4) // NW
CHUNK = NPW // 2


def _minmax_body(x_ref, mn_ref, mx_ref):
    x = x_ref[...]
    mn_ref[...] = jnp.full((1, 1, 128), jnp.min(x), jnp.float32)
    mx_ref[...] = jnp.full((1, 1, 128), jnp.max(x), jnp.float32)


def _minmax(x):
    return pl.pallas_call(
        _minmax_body,
        grid=(B,),
        in_specs=[pl.BlockSpec((1, C, H, W), lambda i: (i, 0, 0, 0))],
        out_specs=[
            pl.BlockSpec((1, 1, 128), lambda i: (i, 0, 0)),
            pl.BlockSpec((1, 1, 128), lambda i: (i, 0, 0)),
        ],
        out_shape=[
            jax.ShapeDtypeStruct((B, 1, 128), jnp.float32),
            jax.ShapeDtypeStruct((B, 1, 128), jnp.float32),
        ],
    )(x)


def _joint_body(x_ref, m_ref, mn_ref, mx_ref, o_ref):
    mn = mn_ref[0][:, 0:1]
    mx = mx_ref[0][:, 0:1]
    d = (mx - mn) + jnp.float32(1e-9)
    x = x_ref[0]
    nr = (x[0] - mn) / d
    ng = (x[1] - mn) / d
    nb = (x[2] - mn) / d
    gray = nr * jnp.float32(0.299) + ng * jnp.float32(0.587)
    gray = gray + nb * jnp.float32(0.114)
    q = (gray * jnp.float32(255.0)).astype(jnp.int32)
    m = m_ref[0]
    mm = jnp.maximum(jnp.maximum(jnp.maximum(jnp.maximum(m[0], m[1]), m[2]), m[3]), m[4])
    cls = jnp.full(q.shape, 4, jnp.int32)
    for c in (3, 2, 1, 0):
        cls = jnp.where(m[c] == mm, jnp.int32(c), cls)
    joint = cls * NUM_BINS + q
    o_ref[0, 0:512] = jnp.bitwise_or(
        joint[:, 0:128], lax.shift_left(joint[:, 128:256], 16))
    o_ref[0, 512:1024] = jnp.bitwise_or(
        joint[:, 256:384], lax.shift_left(joint[:, 384:512], 16))


def _joint_half(x, m, mn, mx, img_base):
    return pl.pallas_call(
        _joint_body,
        grid=(BH,),
        in_specs=[
            pl.BlockSpec((1, C, H, W), lambda i: (img_base + i, 0, 0, 0)),
            pl.BlockSpec((1, NUM_CL, H, W), lambda i: (img_base + i, 0, 0, 0)),
            pl.BlockSpec((1, 1, 128), lambda i: (img_base + i, 0, 0)),
            pl.BlockSpec((1, 1, 128), lambda i: (img_base + i, 0, 0)),
        ],
        out_specs=pl.BlockSpec((1, 1024, 128), lambda i: (i, 0, 0)),
        out_shape=jax.ShapeDtypeStruct((BH, 1024, 128), jnp.int32),
    )(x, m, mn, mx)


def _sc_process(buf, lanehist, laneoff, ones):
    @plsc.parallel_loop(0, CHUNK // 16, 1, unroll=8)
    def _(j):
        v = buf[pl.ds(j * 16, 16)]
        lo = jnp.bitwise_and(v, jnp.int32(0xFFFF))
        hi = lax.shift_right_logical(v, 16)
        plsc.addupdate_scatter(lanehist, [lo + laneoff], ones)
        plsc.addupdate_scatter(lanehist, [hi + laneoff], ones)


def _sc_hist_body(joint_hbm, out_hbm, buf0, buf1, lanehist, outbuf,
                  sem0, sem1, *, base0):
    wid = lax.axis_index("s") * SC_NC + lax.axis_index("c")
    start = base0 + wid * NPW
    laneoff = lax.iota(jnp.int32, 16) * NBINS_J
    ones = jnp.ones((16,), jnp.int32)
    zeros16 = jnp.zeros((16,), jnp.int32)

    c0 = pltpu.async_copy(joint_hbm.at[pl.ds(start, CHUNK)], buf0, sem0)
    c1 = pltpu.async_copy(joint_hbm.at[pl.ds(start + CHUNK, CHUNK)], buf1, sem1)

    @plsc.parallel_loop(0, (16 * NBINS_J) // 16, 1, unroll=8)
    def _(i):
        lanehist[pl.ds(i * 16, 16)] = zeros16

    c0.wait()
    _sc_process(buf0, lanehist, laneoff, ones)
    c1.wait()
    _sc_process(buf1, lanehist, laneoff, ones)

    @plsc.parallel_loop(0, NBINS_J // 16, 1, unroll=2)
    def _(i):
        acc = lanehist[pl.ds(i * 16, 16)]
        for l in range(1, 16):
            acc = acc + lanehist[pl.ds(l * NBINS_J + i * 16, 16)]
        outbuf[pl.ds(i * 16, 16)] = acc
    pltpu.sync_copy(outbuf, out_hbm.at[wid])


def _sc_hist(joint_flat, base0):
    mesh = plsc.VectorSubcoreMesh(
        core_axis_name="c", subcore_axis_name="s",
        num_cores=SC_NC, num_subcores=SC_NS,
    )
    f = pl.kernel(
        functools.partial(_sc_hist_body, base0=base0),
        out_type=jax.ShapeDtypeStruct((NW, NBINS_J), jnp.int32),
        mesh=mesh,
        scratch_types=[
            pltpu.VMEM((CHUNK,), jnp.int32),
            pltpu.VMEM((CHUNK,), jnp.int32),
            pltpu.VMEM((16 * NBINS_J,), jnp.int32),
            pltpu.VMEM((NBINS_J,), jnp.int32),
            pltpu.SemaphoreType.DMA,
            pltpu.SemaphoreType.DMA,
        ],
        compiler_params=pltpu.CompilerParams(needs_layout_passes=False),
    )
    return f(joint_flat)


def _final_body(pa_ref, pb_ref, pj_ref, pi_ref, pm_ref, pim_ref, pmi_ref):
    com_i = jnp.sum(pa_ref[...], axis=0) + jnp.sum(pb_ref[...], axis=0)
    zeros_cnt = jnp.sum(com_i[:, 0:1])
    col = lax.broadcasted_iota(jnp.int32, (NUM_CL, NUM_BINS), 1)
    com_i = jnp.where(col == 0, zeros_cnt, com_i)
    p_joint = com_i.astype(jnp.float32) / jnp.float32(NPIX)
    p_img = jnp.sum(p_joint, axis=0, keepdims=True)
    p_mask = jnp.sum(p_joint, axis=1, keepdims=True)
    eps = jnp.float32(1e-9)
    pj_ref[...] = p_joint
    pi_ref[...] = p_img
    pm_ref[...] = p_mask
    pim_ref[...] = p_joint / (p_mask + eps)
    pmi_ref[...] = p_joint / (p_img + eps)


def _finalize(parts_a, parts_b):
    return pl.pallas_call(
        _final_body,
        out_shape=[
            jax.ShapeDtypeStruct((NUM_CL, NUM_BINS), jnp.float32),
            jax.ShapeDtypeStruct((1, NUM_BINS), jnp.float32),
            jax.ShapeDtypeStruct((NUM_CL, 1), jnp.float32),
            jax.ShapeDtypeStruct((NUM_CL, NUM_BINS), jnp.float32),
            jax.ShapeDtypeStruct((NUM_CL, NUM_BINS), jnp.float32),
        ],
    )(parts_a, parts_b)


def kernel(input, mask):
    mn, mx = _minmax(input)
    joint_a = _joint_half(input, mask, mn, mx, 0)
    parts_a = _sc_hist(joint_a.reshape(NPIX // 4), 0)
    joint_b = _joint_half(input, mask, mn, mx, BH)
    parts_b = _sc_hist(joint_b.reshape(NPIX // 4), 0)
    p_joint, p_img, p_mask, p_img_mask, p_mask_img = _finalize(
        parts_a.reshape(NW, NUM_CL, NUM_BINS),
        parts_b.reshape(NW, NUM_CL, NUM_BINS),
    )
    return (p_joint, p_img.reshape(NUM_BINS), p_mask.reshape(NUM_CL),
            p_img_mask, p_mask_img)
